# Initial kernel scaffold; baseline (speedup 1.0000x reference)
import jax
import jax.numpy as jnp
from jax.experimental import pallas as pl


def kernel(inputs, table):
    raise NotImplementedError("write your pallas kernel here")


# speedup vs baseline: 2.4463x; 2.4463x over previous
import functools

import jax
import jax.numpy as jnp
from jax import lax
from jax.experimental import pallas as pl
from jax.experimental.pallas import tpu as pltpu
from jax.experimental.pallas import tpu_sc as plsc

VOCAB = 1000000
EMBED_DIM = 32
BATCH = 16384
SEQ = 50

NC = 2
NS = 16
NW = NC * NS
LANES = 16

ROWS_PER_W = BATCH // NW
CHUNK = 16
NIDX = CHUNK * SEQ
NCHUNKS = ROWS_PER_W // CHUNK

_MESH = plsc.VectorSubcoreMesh(
    core_axis_name="c", subcore_axis_name="s", num_cores=NC, num_subcores=NS
)


@functools.partial(
    pl.kernel,
    out_type=jax.ShapeDtypeStruct((BATCH, EMBED_DIM), jnp.float32),
    mesh=_MESH,
    scratch_types=[
        pltpu.VMEM((NIDX,), jnp.int32),
        pltpu.VMEM((NIDX, EMBED_DIM), jnp.float32),
        pltpu.VMEM((CHUNK, EMBED_DIM), jnp.float32),
        pltpu.SemaphoreType.DMA,
    ],
    compiler_params=pltpu.CompilerParams(use_tc_tiling_on_sc=False),
)
def _embed_bag(idx_hbm, table_hbm, out_hbm, idx_v, rows_v, out_v, sem):
    wid = lax.axis_index("s") * NC + lax.axis_index("c")
    scale = jnp.float32(1.0 / SEQ)

    def chunk_body(c, _):
        base_b = wid * ROWS_PER_W + c * CHUNK
        pltpu.sync_copy(idx_hbm.at[pl.ds(base_b * SEQ, NIDX)], idx_v)
        pltpu.async_copy(table_hbm.at[idx_v], rows_v, sem).wait()

        def row_body(b, _):
            def acc_body(l, accs):
                a0, a1 = accs
                r = b * SEQ + l
                return (a0 + rows_v[r, pl.ds(0, LANES)],
                        a1 + rows_v[r, pl.ds(LANES, LANES)])

            z = jnp.zeros((LANES,), jnp.float32)
            a0, a1 = lax.fori_loop(0, SEQ, acc_body, (z, z))
            out_v[b, pl.ds(0, LANES)] = a0 * scale
            out_v[b, pl.ds(LANES, LANES)] = a1 * scale
            return 0

        lax.fori_loop(0, CHUNK, row_body, 0)
        pltpu.sync_copy(out_v, out_hbm.at[pl.ds(base_b, CHUNK)])
        return 0

    lax.fori_loop(0, NCHUNKS, chunk_body, 0)


def kernel(inputs, table):
    idx_flat = inputs.reshape(-1).astype(jnp.int32)
    return _embed_bag(idx_flat, table)


# speedup vs baseline: 3.0061x; 1.2288x over previous
import functools

import jax
import jax.numpy as jnp
from jax import lax
from jax.experimental import pallas as pl
from jax.experimental.pallas import tpu as pltpu
from jax.experimental.pallas import tpu_sc as plsc

VOCAB = 1000000
EMBED_DIM = 32
BATCH = 16384
SEQ = 50

NC = 2
NS = 16
NW = NC * NS
LANES = 16

ROWS_PER_W = BATCH // NW

_MESH = plsc.VectorSubcoreMesh(
    core_axis_name="c", subcore_axis_name="s", num_cores=NC, num_subcores=NS
)


@functools.partial(
    pl.kernel,
    out_type=jax.ShapeDtypeStruct((BATCH, EMBED_DIM), jnp.float32),
    mesh=_MESH,
    scratch_types=[
        pltpu.VMEM((SEQ, ROWS_PER_W), jnp.int32),
        pltpu.VMEM((ROWS_PER_W, EMBED_DIM), jnp.float32),
        pltpu.SemaphoreType.DMA,
        pltpu.SemaphoreType.DMA,
    ],
    compiler_params=pltpu.CompilerParams(use_tc_tiling_on_sc=False),
)
def _embed_bag(idx_hbm, table_hbm, out_hbm, idx_v, acc_v, sem0, sem1):
    wid = lax.axis_index("s") * NC + lax.axis_index("c")
    base_b = wid * ROWS_PER_W
    scale = jnp.float32(1.0 / SEQ)

    pltpu.sync_copy(idx_hbm.at[:, pl.ds(base_b, ROWS_PER_W)], idx_v)

    pltpu.async_copy(table_hbm.at[idx_v.at[0]], acc_v, sem0).wait()
    for l in range(1, SEQ):
        pltpu.async_copy(table_hbm.at[idx_v.at[l]], acc_v, sem1, add=True)
    pltpu.make_async_copy(table_hbm.at[idx_v.at[1]], acc_v, sem1).wait()
    for l in range(2, SEQ):
        pltpu.make_async_copy(table_hbm.at[idx_v.at[l]], acc_v, sem1).wait()

    def scale_body(b, _):
        acc_v[b, pl.ds(0, LANES)] = acc_v[b, pl.ds(0, LANES)] * scale
        acc_v[b, pl.ds(LANES, LANES)] = acc_v[b, pl.ds(LANES, LANES)] * scale
        return 0

    lax.fori_loop(0, ROWS_PER_W, scale_body, 0)
    pltpu.sync_copy(acc_v, out_hbm.at[pl.ds(base_b, ROWS_PER_W)])


def kernel(inputs, table):
    idx_t = inputs.astype(jnp.int32).T
    return _embed_bag(idx_t, table)
